# baseline (speedup 1.0000x reference)
import functools

import jax
import jax.numpy as jnp
import numpy as np
from jax import lax
from jax.experimental import pallas as pl
from jax.experimental.pallas import tpu as pltpu
from jax.experimental.pallas import tpu_sc as plsc

NH, NL, NP, C = 8, 4, 4, 256
HD = C // NH
BS = 2
NQ = 5440
NROWS = BS * NQ * NH
NLANE = NH * NL * NP
NCORNER = 4

QT = 320
NQT = NQ // QT

NTILES = 32
QPT = BS * NQ // NTILES
BLK = 20
NBLK = QPT // BLK
NGR = NCORNER * NLANE


def _np_selectors():
    lane = np.arange(NLANE)
    lvl = (lane // NP) % NL
    wl_i = 64 >> lvl
    rpsel = np.zeros((NL * 2, 2 * NLANE), np.float32)
    for l in range(NL):
        sel_lanes = np.nonzero(lvl == l)[0]
        rpsel[2 * l + 0, sel_lanes] = 1.0
        rpsel[2 * l + 1, NLANE + sel_lanes] = 1.0
    grp = lane // (NL * NP)
    bmask = (grp[:, None] == grp[None, :]).astype(np.float32)
    lsi = (16384 - 4 * wl_i * wl_i) // 3
    hidx = lane // (NL * NP)
    cst = np.stack([wl_i, lsi, hidx]).astype(np.int32)
    return rpsel, bmask, cst


def _prep_body(q_ref, rp_ref, wall_ref, ball_ref, sel_ref, bmask_ref, cst_ref,
               val_ref, i0_ref, i1_ref, i2_ref, i3_ref,
               w0_ref, w1_ref, w2_ref, w3_ref):
    b = pl.program_id(0)
    q = q_ref[...]
    res = jnp.dot(q, wall_ref[...], preferred_element_type=jnp.float32)
    res = res + ball_ref[...]
    val_ref[...] = res[:, :C].astype(jnp.bfloat16)
    xoff = res[:, C:C + NLANE]
    yoff = res[:, C + NLANE:C + 2 * NLANE]
    logits = res[:, C + 2 * NLANE:]

    m = jnp.max(logits, axis=-1, keepdims=True)
    e = jnp.exp(logits - m)
    e_hi = e.astype(jnp.bfloat16).astype(jnp.float32)
    s = (jnp.dot(e_hi, bmask_ref[...], preferred_element_type=jnp.float32)
         + jnp.dot(e - e_hi, bmask_ref[...],
                   preferred_element_type=jnp.float32))
    attn = e / s

    rp = rp_ref[...]
    rp_hi = rp.astype(jnp.bfloat16).astype(jnp.float32)
    rpb = (jnp.dot(rp_hi, sel_ref[...], preferred_element_type=jnp.float32)
           + jnp.dot(rp - rp_hi, sel_ref[...],
                     preferred_element_type=jnp.float32))

    wl_i = cst_ref[0:1, :]
    lsi = cst_ref[1:2, :]
    hidx = cst_ref[2:3, :]
    wl = wl_i.astype(jnp.float32)

    x = rpb[:, :NLANE] * wl + xoff
    y = rpb[:, NLANE:] * wl + yoff
    x0 = jnp.floor(x)
    y0 = jnp.floor(y)
    wx = x - x0
    wy = y - y0

    row_base = (b * NQ + lsi) * NH + hidx

    corner_refs = ((i0_ref, w0_ref), (i1_ref, w1_ref),
                   (i2_ref, w2_ref), (i3_ref, w3_ref))
    for (dx, dy, wfac), (i_ref, w_ref) in zip((
        (0.0, 0.0, (1 - wx) * (1 - wy)),
        (1.0, 0.0, wx * (1 - wy)),
        (0.0, 1.0, (1 - wx) * wy),
        (1.0, 1.0, wx * wy),
    ), corner_refs):
        xi = x0 + dx
        yi = y0 + dy
        valid = ((xi >= 0) & (xi <= wl - 1) & (yi >= 0) & (yi <= wl - 1))
        xc = jnp.clip(xi, 0, wl - 1).astype(jnp.int32)
        yc = jnp.clip(yi, 0, wl - 1).astype(jnp.int32)
        i_ref[...] = row_base + (yc * wl_i + xc) * NH
        w_ref[...] = attn * wfac * valid.astype(jnp.float32)


def _post_body(slo_ref, shi_ref, q_ref, wlo_ref, whi_ref, b_ref, o_ref):
    o_ref[...] = (jnp.dot(slo_ref[...], wlo_ref[...],
                          preferred_element_type=jnp.float32)
                  + jnp.dot(shi_ref[...], whi_ref[...],
                            preferred_element_type=jnp.float32)
                  + b_ref[...] + q_ref[...])


def _sc_gather(table, idxs, wgts):
    mesh = plsc.VectorSubcoreMesh(core_axis_name="c", subcore_axis_name="s")

    @functools.partial(
        pl.kernel,
        mesh=mesh,
        out_type=[jax.ShapeDtypeStruct((BS * NQ, NH // 2 * HD), jnp.float32),
                  jax.ShapeDtypeStruct((BS * NQ, NH // 2 * HD), jnp.float32)],
        compiler_params=pltpu.CompilerParams(use_tc_tiling_on_sc=False,
                                             needs_layout_passes=False),
        scratch_types=[
            pltpu.VMEM((2, NCORNER, BLK, NLANE), jnp.int32),
            pltpu.VMEM((2, NCORNER, BLK, NLANE), jnp.float32),
            pltpu.VMEM((2, NGR, HD), jnp.bfloat16),
            pltpu.VMEM((2, BLK, 2, NH // 2 * HD), jnp.float32),
            pltpu.SemaphoreType.DMA,
            pltpu.SemaphoreType.DMA,
            pltpu.SemaphoreType.DMA,
            pltpu.SemaphoreType.DMA,
        ],
    )
    def k(table_hbm, i0, i1, i2, i3, w0, w1, w2, w3, olo_hbm, ohi_hbm,
          ib_v, wb_v, rows_v, out_v, sem_in, sem_g0, sem_g1, sem_out):
        idx_hbms = (i0, i1, i2, i3)
        wgt_hbms = (w0, w1, w2, w3)
        wid = lax.axis_index("s") * 2 + lax.axis_index("c")
        qbase = wid * QPT

        def fetch_block(blk, buf):
            qs = qbase + blk * BLK
            for c in range(NCORNER):
                pltpu.async_copy(idx_hbms[c].at[pl.ds(qs, BLK)],
                                 ib_v.at[buf, c], sem_in)
                pltpu.async_copy(wgt_hbms[c].at[pl.ds(qs, BLK)],
                                 wb_v.at[buf, c], sem_in)

        def wait_block(buf):
            for c in range(NCORNER):
                pltpu.make_async_copy(
                    idx_hbms[c].at[pl.ds(0, BLK)], ib_v.at[buf, c],
                    sem_in).wait()
                pltpu.make_async_copy(
                    wgt_hbms[c].at[pl.ds(0, BLK)], wb_v.at[buf, c],
                    sem_in).wait()

        def fire(pb, jq, p, sem):
            for c in range(NCORNER):
                pltpu.async_copy(
                    table_hbm.at[ib_v.at[pb, c, jq]],
                    rows_v.at[p, pl.ds(c * NLANE, NLANE)], sem)

        def drain(p, sem):
            pltpu.make_async_copy(
                table_hbm.at[pl.ds(0, NGR)], rows_v.at[p], sem).wait()

        def put_block(blk, buf):
            qs = qbase + blk * BLK
            pltpu.async_copy(out_v.at[buf, slice(None), 0],
                             olo_hbm.at[pl.ds(qs, BLK)], sem_out)
            pltpu.async_copy(out_v.at[buf, slice(None), 1],
                             ohi_hbm.at[pl.ds(qs, BLK)], sem_out)

        def drain_out():
            pltpu.make_async_copy(
                out_v.at[0, slice(None), 0],
                olo_hbm.at[pl.ds(0, BLK)], sem_out).wait()
            pltpu.make_async_copy(
                out_v.at[0, slice(None), 1],
                ohi_hbm.at[pl.ds(0, BLK)], sem_out).wait()

        def compute(pb, jq, p):
            def h_body(hh, _):
                for u in range(2):
                    h = hh * 2 + u
                    a0 = jnp.zeros((16,), jnp.float32)
                    a1 = jnp.zeros((16,), jnp.float32)
                    for c in range(NCORNER):
                        wv = wb_v[pb, c, jq, pl.ds(h * 16, 16)]
                        for j in range(16):
                            w = wv[j]
                            r = c * NLANE + h * 16 + j
                            ev, od = plsc.unpack(
                                rows_v[p, r, pl.ds(0, 32)],
                                format=plsc.PackFormat.INTERLEAVED)
                            a0 = a0 + w * ev
                            a1 = a1 + w * od
                    half = h // (NH // 2)
                    off = (h % (NH // 2)) * HD
                    out_v[pb, jq, half, pl.ds(off, 16)] = a0
                    out_v[pb, jq, half, pl.ds(off + 16, 16)] = a1
                return 0

            lax.fori_loop(0, NH // 2, h_body, 0)

        fetch_block(0, 0)
        wait_block(0)
        fire(0, 0, 0, sem_g0)

        def blk_body(B, _):
            pb = B & 1

            @pl.when(B < NBLK - 1)
            def _():
                fetch_block(B + 1, 1 - pb)

            @pl.when(B >= 2)
            def _():
                drain_out()

            def pair_body(kk, _):
                ja = 2 * kk
                fire(pb, ja + 1, 1, sem_g1)
                drain(0, sem_g0)
                compute(pb, ja, 0)

                @pl.when(kk < BLK // 2 - 1)
                def _():
                    fire(pb, ja + 2, 0, sem_g0)

                @pl.when(kk == BLK // 2 - 1)
                def _():
                    @pl.when(B < NBLK - 1)
                    def _():
                        wait_block(1 - pb)
                        fire(1 - pb, 0, 0, sem_g0)

                drain(1, sem_g1)
                compute(pb, ja + 1, 1)
                return 0

            lax.fori_loop(0, BLK // 2, pair_body, 0)
            put_block(B, pb)
            return 0

        lax.fori_loop(0, NBLK, blk_body, 0)
        drain_out()
        drain_out()

    return k(table, *idxs, *wgts)


def kernel(query, reference_points, spatial_shapes, level_start_index,
           W_off, b_off, W_attn, b_attn, W_val, b_val, W_out, b_out):
    q2 = query.reshape(BS * NQ, C)
    rp2 = reference_points.reshape(BS * NQ, NL * 2)
    rpsel_np, bmask_np, cst_np = _np_selectors()
    wall = jnp.concatenate(
        [W_val, W_off[0::2], W_off[1::2], W_attn], axis=0).T
    half = jnp.float32(0.5)
    ball = jnp.concatenate(
        [b_val, b_off[0::2] - half, b_off[1::2] - half, b_attn]).reshape(1, -1)
    sel = jnp.asarray(rpsel_np)
    bmask = jnp.asarray(bmask_np)
    cst = jnp.asarray(cst_np)

    prep_out = pl.pallas_call(
        _prep_body,
        grid=(BS, NQT),
        in_specs=[
            pl.BlockSpec((QT, C), lambda b, i: (b * NQT + i, 0)),
            pl.BlockSpec((QT, NL * 2), lambda b, i: (b * NQT + i, 0)),
            pl.BlockSpec((C, C + 3 * NLANE), lambda b, i: (0, 0)),
            pl.BlockSpec((1, C + 3 * NLANE), lambda b, i: (0, 0)),
            pl.BlockSpec((NL * 2, 2 * NLANE), lambda b, i: (0, 0)),
            pl.BlockSpec((NLANE, NLANE), lambda b, i: (0, 0)),
            pl.BlockSpec((3, NLANE), lambda b, i: (0, 0)),
        ],
        out_specs=[pl.BlockSpec((QT, C), lambda b, i: (b * NQT + i, 0))]
        + [pl.BlockSpec((QT, NLANE), lambda b, i: (b * NQT + i, 0))] * 8,
        out_shape=[jax.ShapeDtypeStruct((BS * NQ, C), jnp.bfloat16)]
        + [jax.ShapeDtypeStruct((BS * NQ, NLANE), jnp.int32)] * 4
        + [jax.ShapeDtypeStruct((BS * NQ, NLANE), jnp.float32)] * 4,
    )(q2, rp2, wall, ball, sel, bmask, cst)
    value = prep_out[0]
    idxs = prep_out[1:5]
    wgts = prep_out[5:9]

    s_lo, s_hi = _sc_gather(value.reshape(NROWS, HD), idxs, wgts)

    perm = np.concatenate([np.concatenate([np.arange(h * HD, (h + 1) * HD, 2),
                                           np.arange(h * HD + 1, (h + 1) * HD, 2)])
                           for h in range(NH)])
    w_out_t = W_out.T[perm, :]

    out = pl.pallas_call(
        _post_body,
        grid=(BS * NQT,),
        in_specs=[
            pl.BlockSpec((QT, C // 2), lambda i: (i, 0)),
            pl.BlockSpec((QT, C // 2), lambda i: (i, 0)),
            pl.BlockSpec((QT, C), lambda i: (i, 0)),
            pl.BlockSpec((C // 2, C), lambda i: (0, 0)),
            pl.BlockSpec((C // 2, C), lambda i: (0, 0)),
            pl.BlockSpec((1, C), lambda i: (0, 0)),
        ],
        out_specs=pl.BlockSpec((QT, C), lambda i: (i, 0)),
        out_shape=jax.ShapeDtypeStruct((BS * NQ, C), jnp.float32),
    )(s_lo, s_hi, q2, w_out_t[:C // 2], w_out_t[C // 2:],
      b_out.reshape(1, C))

    return out.reshape(BS, NQ, C)


# speedup vs baseline: 1.0059x; 1.0059x over previous
import functools

import jax
import jax.numpy as jnp
import numpy as np
from jax import lax
from jax.experimental import pallas as pl
from jax.experimental.pallas import tpu as pltpu
from jax.experimental.pallas import tpu_sc as plsc

NH, NL, NP, C = 8, 4, 4, 256
HD = C // NH
BS = 2
NQ = 5440
NROWS = BS * NQ * NH
NLANE = NH * NL * NP
NCORNER = 4

QT = 320
NQT = NQ // QT

NTILES = 32
QPT = BS * NQ // NTILES
BLK = 10
NBLK = QPT // BLK
NGR = NCORNER * NLANE


def _np_selectors():
    lane = np.arange(NLANE)
    lvl = (lane // NP) % NL
    wl_i = 64 >> lvl
    rpsel = np.zeros((NL * 2, 2 * NLANE), np.float32)
    for l in range(NL):
        sel_lanes = np.nonzero(lvl == l)[0]
        rpsel[2 * l + 0, sel_lanes] = 1.0
        rpsel[2 * l + 1, NLANE + sel_lanes] = 1.0
    grp = lane // (NL * NP)
    bmask = (grp[:, None] == grp[None, :]).astype(np.float32)
    lsi = (16384 - 4 * wl_i * wl_i) // 3
    hidx = lane // (NL * NP)
    cst = np.stack([wl_i, lsi, hidx]).astype(np.int32)
    return rpsel, bmask, cst


def _prep_body(q_ref, rp_ref, wall_ref, ball_ref, sel_ref, bmask_ref, cst_ref,
               val_ref, i0_ref, i1_ref, i2_ref, i3_ref,
               w0_ref, w1_ref, w2_ref, w3_ref):
    b = pl.program_id(0)
    q = q_ref[...]
    res = jnp.dot(q, wall_ref[...], preferred_element_type=jnp.float32)
    res = res + ball_ref[...]
    val_ref[...] = res[:, :C].astype(jnp.bfloat16)
    xoff = res[:, C:C + NLANE]
    yoff = res[:, C + NLANE:C + 2 * NLANE]
    logits = res[:, C + 2 * NLANE:]

    m = jnp.max(logits, axis=-1, keepdims=True)
    e = jnp.exp(logits - m)
    e_hi = e.astype(jnp.bfloat16).astype(jnp.float32)
    s = (jnp.dot(e_hi, bmask_ref[...], preferred_element_type=jnp.float32)
         + jnp.dot(e - e_hi, bmask_ref[...],
                   preferred_element_type=jnp.float32))
    attn = e / s

    rp = rp_ref[...]
    rp_hi = rp.astype(jnp.bfloat16).astype(jnp.float32)
    rpb = (jnp.dot(rp_hi, sel_ref[...], preferred_element_type=jnp.float32)
           + jnp.dot(rp - rp_hi, sel_ref[...],
                     preferred_element_type=jnp.float32))

    wl_i = cst_ref[0:1, :]
    lsi = cst_ref[1:2, :]
    hidx = cst_ref[2:3, :]
    wl = wl_i.astype(jnp.float32)

    x = rpb[:, :NLANE] * wl + xoff
    y = rpb[:, NLANE:] * wl + yoff
    x0 = jnp.floor(x)
    y0 = jnp.floor(y)
    wx = x - x0
    wy = y - y0

    row_base = (b * NQ + lsi) * NH + hidx

    corner_refs = ((i0_ref, w0_ref), (i1_ref, w1_ref),
                   (i2_ref, w2_ref), (i3_ref, w3_ref))
    for (dx, dy, wfac), (i_ref, w_ref) in zip((
        (0.0, 0.0, (1 - wx) * (1 - wy)),
        (1.0, 0.0, wx * (1 - wy)),
        (0.0, 1.0, (1 - wx) * wy),
        (1.0, 1.0, wx * wy),
    ), corner_refs):
        xi = x0 + dx
        yi = y0 + dy
        valid = ((xi >= 0) & (xi <= wl - 1) & (yi >= 0) & (yi <= wl - 1))
        xc = jnp.clip(xi, 0, wl - 1).astype(jnp.int32)
        yc = jnp.clip(yi, 0, wl - 1).astype(jnp.int32)
        i_ref[...] = row_base + (yc * wl_i + xc) * NH
        w_ref[...] = attn * wfac * valid.astype(jnp.float32)


def _post_body(slo_ref, shi_ref, q_ref, wlo_ref, whi_ref, b_ref, o_ref):
    o_ref[...] = (jnp.dot(slo_ref[...], wlo_ref[...],
                          preferred_element_type=jnp.float32)
                  + jnp.dot(shi_ref[...], whi_ref[...],
                            preferred_element_type=jnp.float32)
                  + b_ref[...] + q_ref[...])


def _sc_gather(table, idxs, wgts):
    mesh = plsc.VectorSubcoreMesh(core_axis_name="c", subcore_axis_name="s")

    @functools.partial(
        pl.kernel,
        mesh=mesh,
        out_type=[jax.ShapeDtypeStruct((BS * NQ, NH // 2 * HD), jnp.float32),
                  jax.ShapeDtypeStruct((BS * NQ, NH // 2 * HD), jnp.float32)],
        compiler_params=pltpu.CompilerParams(use_tc_tiling_on_sc=False,
                                             needs_layout_passes=False),
        scratch_types=[
            pltpu.VMEM((2, NCORNER, BLK, NLANE), jnp.int32),
            pltpu.VMEM((2, NCORNER, BLK, NLANE), jnp.float32),
            pltpu.VMEM((2, NGR, HD), jnp.bfloat16),
            pltpu.VMEM((2, BLK, 2, NH // 2 * HD), jnp.float32),
            pltpu.SemaphoreType.DMA,
            pltpu.SemaphoreType.DMA,
            pltpu.SemaphoreType.DMA,
            pltpu.SemaphoreType.DMA,
        ],
    )
    def k(table_hbm, i0, i1, i2, i3, w0, w1, w2, w3, olo_hbm, ohi_hbm,
          ib_v, wb_v, rows_v, out_v, sem_in, sem_g0, sem_g1, sem_out):
        idx_hbms = (i0, i1, i2, i3)
        wgt_hbms = (w0, w1, w2, w3)
        wid = lax.axis_index("s") * 2 + lax.axis_index("c")
        qbase = wid * QPT

        def fetch_block(blk, buf):
            qs = qbase + blk * BLK
            for c in range(NCORNER):
                pltpu.async_copy(idx_hbms[c].at[pl.ds(qs, BLK)],
                                 ib_v.at[buf, c], sem_in)
                pltpu.async_copy(wgt_hbms[c].at[pl.ds(qs, BLK)],
                                 wb_v.at[buf, c], sem_in)

        def wait_block(buf):
            for c in range(NCORNER):
                pltpu.make_async_copy(
                    idx_hbms[c].at[pl.ds(0, BLK)], ib_v.at[buf, c],
                    sem_in).wait()
                pltpu.make_async_copy(
                    wgt_hbms[c].at[pl.ds(0, BLK)], wb_v.at[buf, c],
                    sem_in).wait()

        def fire(pb, jq, p, sem):
            for c in range(NCORNER):
                pltpu.async_copy(
                    table_hbm.at[ib_v.at[pb, c, jq]],
                    rows_v.at[p, pl.ds(c * NLANE, NLANE)], sem)

        def drain(p, sem):
            pltpu.make_async_copy(
                table_hbm.at[pl.ds(0, NGR)], rows_v.at[p], sem).wait()

        def put_block(blk, buf):
            qs = qbase + blk * BLK
            pltpu.async_copy(out_v.at[buf, slice(None), 0],
                             olo_hbm.at[pl.ds(qs, BLK)], sem_out)
            pltpu.async_copy(out_v.at[buf, slice(None), 1],
                             ohi_hbm.at[pl.ds(qs, BLK)], sem_out)

        def drain_out():
            pltpu.make_async_copy(
                out_v.at[0, slice(None), 0],
                olo_hbm.at[pl.ds(0, BLK)], sem_out).wait()
            pltpu.make_async_copy(
                out_v.at[0, slice(None), 1],
                ohi_hbm.at[pl.ds(0, BLK)], sem_out).wait()

        def compute(pb, jq, p):
            def h_body(hh, _):
                for u in range(2):
                    h = hh * 2 + u
                    a0 = jnp.zeros((16,), jnp.float32)
                    a1 = jnp.zeros((16,), jnp.float32)
                    for c in range(NCORNER):
                        wv = wb_v[pb, c, jq, pl.ds(h * 16, 16)]
                        for j in range(16):
                            w = wv[j]
                            r = c * NLANE + h * 16 + j
                            ev, od = plsc.unpack(
                                rows_v[p, r, pl.ds(0, 32)],
                                format=plsc.PackFormat.INTERLEAVED)
                            a0 = a0 + w * ev
                            a1 = a1 + w * od
                    half = h // (NH // 2)
                    off = (h % (NH // 2)) * HD
                    out_v[pb, jq, half, pl.ds(off, 16)] = a0
                    out_v[pb, jq, half, pl.ds(off + 16, 16)] = a1
                return 0

            lax.fori_loop(0, NH // 2, h_body, 0)

        fetch_block(0, 0)
        wait_block(0)
        fire(0, 0, 0, sem_g0)

        def blk_body(B, _):
            pb = B & 1

            @pl.when(B < NBLK - 1)
            def _():
                fetch_block(B + 1, 1 - pb)

            @pl.when(B >= 2)
            def _():
                drain_out()

            def pair_body(kk, _):
                ja = 2 * kk
                fire(pb, ja + 1, 1, sem_g1)
                drain(0, sem_g0)
                compute(pb, ja, 0)

                @pl.when(kk < BLK // 2 - 1)
                def _():
                    fire(pb, ja + 2, 0, sem_g0)

                @pl.when(kk == BLK // 2 - 1)
                def _():
                    @pl.when(B < NBLK - 1)
                    def _():
                        wait_block(1 - pb)
                        fire(1 - pb, 0, 0, sem_g0)

                drain(1, sem_g1)
                compute(pb, ja + 1, 1)
                return 0

            lax.fori_loop(0, BLK // 2, pair_body, 0)
            put_block(B, pb)
            return 0

        lax.fori_loop(0, NBLK, blk_body, 0)
        drain_out()
        drain_out()

    return k(table, *idxs, *wgts)


def kernel(query, reference_points, spatial_shapes, level_start_index,
           W_off, b_off, W_attn, b_attn, W_val, b_val, W_out, b_out):
    q2 = query.reshape(BS * NQ, C)
    rp2 = reference_points.reshape(BS * NQ, NL * 2)
    rpsel_np, bmask_np, cst_np = _np_selectors()
    wall = jnp.concatenate(
        [W_val, W_off[0::2], W_off[1::2], W_attn], axis=0).T
    half = jnp.float32(0.5)
    ball = jnp.concatenate(
        [b_val, b_off[0::2] - half, b_off[1::2] - half, b_attn]).reshape(1, -1)
    sel = jnp.asarray(rpsel_np)
    bmask = jnp.asarray(bmask_np)
    cst = jnp.asarray(cst_np)

    prep_out = pl.pallas_call(
        _prep_body,
        grid=(BS, NQT),
        in_specs=[
            pl.BlockSpec((QT, C), lambda b, i: (b * NQT + i, 0)),
            pl.BlockSpec((QT, NL * 2), lambda b, i: (b * NQT + i, 0)),
            pl.BlockSpec((C, C + 3 * NLANE), lambda b, i: (0, 0)),
            pl.BlockSpec((1, C + 3 * NLANE), lambda b, i: (0, 0)),
            pl.BlockSpec((NL * 2, 2 * NLANE), lambda b, i: (0, 0)),
            pl.BlockSpec((NLANE, NLANE), lambda b, i: (0, 0)),
            pl.BlockSpec((3, NLANE), lambda b, i: (0, 0)),
        ],
        out_specs=[pl.BlockSpec((QT, C), lambda b, i: (b * NQT + i, 0))]
        + [pl.BlockSpec((QT, NLANE), lambda b, i: (b * NQT + i, 0))] * 8,
        out_shape=[jax.ShapeDtypeStruct((BS * NQ, C), jnp.bfloat16)]
        + [jax.ShapeDtypeStruct((BS * NQ, NLANE), jnp.int32)] * 4
        + [jax.ShapeDtypeStruct((BS * NQ, NLANE), jnp.float32)] * 4,
    )(q2, rp2, wall, ball, sel, bmask, cst)
    value = prep_out[0]
    idxs = prep_out[1:5]
    wgts = prep_out[5:9]

    s_lo, s_hi = _sc_gather(value.reshape(NROWS, HD), idxs, wgts)

    perm = np.concatenate([np.concatenate([np.arange(h * HD, (h + 1) * HD, 2),
                                           np.arange(h * HD + 1, (h + 1) * HD, 2)])
                           for h in range(NH)])
    w_out_t = W_out.T[perm, :]

    out = pl.pallas_call(
        _post_body,
        grid=(BS * NQT,),
        in_specs=[
            pl.BlockSpec((QT, C // 2), lambda i: (i, 0)),
            pl.BlockSpec((QT, C // 2), lambda i: (i, 0)),
            pl.BlockSpec((QT, C), lambda i: (i, 0)),
            pl.BlockSpec((C // 2, C), lambda i: (0, 0)),
            pl.BlockSpec((C // 2, C), lambda i: (0, 0)),
            pl.BlockSpec((1, C), lambda i: (0, 0)),
        ],
        out_specs=pl.BlockSpec((QT, C), lambda i: (i, 0)),
        out_shape=jax.ShapeDtypeStruct((BS * NQ, C), jnp.float32),
    )(s_lo, s_hi, q2, w_out_t[:C // 2], w_out_t[C // 2:],
      b_out.reshape(1, C))

    return out.reshape(BS, NQ, C)


# speedup vs baseline: 1.1063x; 1.0998x over previous
import functools

import jax
import jax.numpy as jnp
import numpy as np
from jax import lax
from jax.experimental import pallas as pl
from jax.experimental.pallas import tpu as pltpu
from jax.experimental.pallas import tpu_sc as plsc

NH, NL, NP, C = 8, 4, 4, 256
HD = C // NH
BS = 2
NQ = 5440
NROWS = BS * NQ * NH
NLANE = NH * NL * NP
NCORNER = 4

QT = 320
NQT = NQ // QT

NTILES = 32
QPT = BS * NQ // NTILES
BLK = 20
NBLK = QPT // BLK
NGR = NCORNER * NLANE


def _np_selectors():
    lane = np.arange(NLANE)
    lvl = (lane // NP) % NL
    wl_i = 64 >> lvl
    rpsel = np.zeros((NL * 2, 2 * NLANE), np.float32)
    for l in range(NL):
        sel_lanes = np.nonzero(lvl == l)[0]
        rpsel[2 * l + 0, sel_lanes] = 1.0
        rpsel[2 * l + 1, NLANE + sel_lanes] = 1.0
    grp = lane // (NL * NP)
    bmask = (grp[:, None] == grp[None, :]).astype(np.float32)
    lsi = (16384 - 4 * wl_i * wl_i) // 3
    hidx = lane // (NL * NP)
    cst = np.stack([wl_i, lsi, hidx]).astype(np.int32)
    return rpsel, bmask, cst


def _prep_body(q_ref, rp_ref, wall_ref, ball_ref, sel_ref, bmask_ref, cst_ref,
               val_ref, i0_ref, i1_ref, i2_ref, i3_ref,
               w0_ref, w1_ref, w2_ref, w3_ref):
    b = pl.program_id(0)
    q = q_ref[...]
    res = jnp.dot(q, wall_ref[...], preferred_element_type=jnp.float32)
    res = res + ball_ref[...]
    val_ref[...] = res[:, :C].astype(jnp.bfloat16)
    xoff = res[:, C:C + NLANE]
    yoff = res[:, C + NLANE:C + 2 * NLANE]
    logits = res[:, C + 2 * NLANE:]

    m = jnp.max(logits, axis=-1, keepdims=True)
    e = jnp.exp(logits - m)
    e_hi = e.astype(jnp.bfloat16).astype(jnp.float32)
    s = (jnp.dot(e_hi, bmask_ref[...], preferred_element_type=jnp.float32)
         + jnp.dot(e - e_hi, bmask_ref[...],
                   preferred_element_type=jnp.float32))
    attn = e / s

    rp = rp_ref[...]
    rp_hi = rp.astype(jnp.bfloat16).astype(jnp.float32)
    rpb = (jnp.dot(rp_hi, sel_ref[...], preferred_element_type=jnp.float32)
           + jnp.dot(rp - rp_hi, sel_ref[...],
                     preferred_element_type=jnp.float32))

    wl_i = cst_ref[0:1, :]
    lsi = cst_ref[1:2, :]
    hidx = cst_ref[2:3, :]
    wl = wl_i.astype(jnp.float32)

    x = rpb[:, :NLANE] * wl + xoff
    y = rpb[:, NLANE:] * wl + yoff
    x0 = jnp.floor(x)
    y0 = jnp.floor(y)
    wx = x - x0
    wy = y - y0

    row_base = (b * NQ + lsi) * NH + hidx

    corner_refs = ((i0_ref, w0_ref), (i1_ref, w1_ref),
                   (i2_ref, w2_ref), (i3_ref, w3_ref))
    for (dx, dy, wfac), (i_ref, w_ref) in zip((
        (0.0, 0.0, (1 - wx) * (1 - wy)),
        (1.0, 0.0, wx * (1 - wy)),
        (0.0, 1.0, (1 - wx) * wy),
        (1.0, 1.0, wx * wy),
    ), corner_refs):
        xi = x0 + dx
        yi = y0 + dy
        valid = ((xi >= 0) & (xi <= wl - 1) & (yi >= 0) & (yi <= wl - 1))
        xc = jnp.clip(xi, 0, wl - 1).astype(jnp.int32)
        yc = jnp.clip(yi, 0, wl - 1).astype(jnp.int32)
        i_ref[...] = row_base + (yc * wl_i + xc) * NH
        w_ref[...] = attn * wfac * valid.astype(jnp.float32)


def _post_body(slo_ref, shi_ref, q_ref, wlo_ref, whi_ref, b_ref, o_ref):
    o_ref[...] = (jnp.dot(slo_ref[...], wlo_ref[...],
                          preferred_element_type=jnp.float32)
                  + jnp.dot(shi_ref[...], whi_ref[...],
                            preferred_element_type=jnp.float32)
                  + b_ref[...] + q_ref[...])


def _sc_gather(table, idxs, wgts):
    mesh = plsc.VectorSubcoreMesh(core_axis_name="c", subcore_axis_name="s")

    @functools.partial(
        pl.kernel,
        mesh=mesh,
        out_type=[jax.ShapeDtypeStruct((BS * NQ, NH // 2 * HD), jnp.float32),
                  jax.ShapeDtypeStruct((BS * NQ, NH // 2 * HD), jnp.float32)],
        compiler_params=pltpu.CompilerParams(use_tc_tiling_on_sc=False,
                                             needs_layout_passes=False),
        scratch_types=[
            pltpu.VMEM((2, NCORNER, BLK, NLANE), jnp.int32),
            pltpu.VMEM((2, NCORNER, BLK, NLANE), jnp.float32),
            pltpu.VMEM((4, NGR, HD), jnp.bfloat16),
            pltpu.VMEM((2, BLK, 2, NH // 2 * HD), jnp.float32),
            pltpu.SemaphoreType.DMA,
            pltpu.SemaphoreType.DMA,
            pltpu.SemaphoreType.DMA,
            pltpu.SemaphoreType.DMA,
            pltpu.SemaphoreType.DMA,
            pltpu.SemaphoreType.DMA,
        ],
    )
    def k(table_hbm, i0, i1, i2, i3, w0, w1, w2, w3, olo_hbm, ohi_hbm,
          ib_v, wb_v, rows_v, out_v, sem_in, sem_g0, sem_g1, sem_g2, sem_g3,
          sem_out):
        idx_hbms = (i0, i1, i2, i3)
        wgt_hbms = (w0, w1, w2, w3)
        wid = lax.axis_index("s") * 2 + lax.axis_index("c")
        qbase = wid * QPT

        def fetch_block(blk, buf):
            qs = qbase + blk * BLK
            for c in range(NCORNER):
                pltpu.async_copy(idx_hbms[c].at[pl.ds(qs, BLK)],
                                 ib_v.at[buf, c], sem_in)
                pltpu.async_copy(wgt_hbms[c].at[pl.ds(qs, BLK)],
                                 wb_v.at[buf, c], sem_in)

        def wait_block(buf):
            for c in range(NCORNER):
                pltpu.make_async_copy(
                    idx_hbms[c].at[pl.ds(0, BLK)], ib_v.at[buf, c],
                    sem_in).wait()
                pltpu.make_async_copy(
                    wgt_hbms[c].at[pl.ds(0, BLK)], wb_v.at[buf, c],
                    sem_in).wait()

        def fire(pb, jq, p, sem):
            for c in range(NCORNER):
                pltpu.async_copy(
                    table_hbm.at[ib_v.at[pb, c, jq]],
                    rows_v.at[p, pl.ds(c * NLANE, NLANE)], sem)

        def drain(p, sem):
            pltpu.make_async_copy(
                table_hbm.at[pl.ds(0, NGR)], rows_v.at[p], sem).wait()

        def put_block(blk, buf):
            qs = qbase + blk * BLK
            pltpu.async_copy(out_v.at[buf, slice(None), 0],
                             olo_hbm.at[pl.ds(qs, BLK)], sem_out)
            pltpu.async_copy(out_v.at[buf, slice(None), 1],
                             ohi_hbm.at[pl.ds(qs, BLK)], sem_out)

        def drain_out():
            pltpu.make_async_copy(
                out_v.at[0, slice(None), 0],
                olo_hbm.at[pl.ds(0, BLK)], sem_out).wait()
            pltpu.make_async_copy(
                out_v.at[0, slice(None), 1],
                ohi_hbm.at[pl.ds(0, BLK)], sem_out).wait()

        def compute(pb, jq, p):
            def h_body(hh, _):
                for u in range(2):
                    h = hh * 2 + u
                    a0 = jnp.zeros((16,), jnp.float32)
                    a1 = jnp.zeros((16,), jnp.float32)
                    for c in range(NCORNER):
                        wv = wb_v[pb, c, jq, pl.ds(h * 16, 16)]
                        for j in range(16):
                            w = wv[j]
                            r = c * NLANE + h * 16 + j
                            ev, od = plsc.unpack(
                                rows_v[p, r, pl.ds(0, 32)],
                                format=plsc.PackFormat.INTERLEAVED)
                            a0 = a0 + w * ev
                            a1 = a1 + w * od
                    half = h // (NH // 2)
                    off = (h % (NH // 2)) * HD
                    out_v[pb, jq, half, pl.ds(off, 16)] = a0
                    out_v[pb, jq, half, pl.ds(off + 16, 16)] = a1
                return 0

            lax.fori_loop(0, NH // 2, h_body, 0)

        sems = (sem_g0, sem_g1, sem_g2, sem_g3)

        fetch_block(0, 0)
        wait_block(0)
        fire(0, 0, 0, sem_g0)
        fire(0, 1, 1, sem_g1)

        def blk_body(B, _):
            pb = B & 1

            @pl.when(B < NBLK - 1)
            def _():
                fetch_block(B + 1, 1 - pb)

            @pl.when(B >= 2)
            def _():
                drain_out()

            def quad_body(t, _):
                for u in range(4):
                    j = 4 * t + u
                    bnext = (u + 2) % 4
                    if u < 2:
                        fire(pb, j + 2, bnext, sems[bnext])
                    else:
                        @pl.when(t < BLK // 4 - 1)
                        def _():
                            fire(pb, j + 2, bnext, sems[bnext])

                        @pl.when(t == BLK // 4 - 1)
                        def _():
                            @pl.when(B < NBLK - 1)
                            def _():
                                if u == 2:
                                    wait_block(1 - pb)
                                fire(1 - pb, u - 2, bnext, sems[bnext])

                    drain(u, sems[u])
                    compute(pb, j, u)
                return 0

            lax.fori_loop(0, BLK // 4, quad_body, 0)
            put_block(B, pb)
            return 0

        lax.fori_loop(0, NBLK, blk_body, 0)
        drain_out()
        drain_out()

    return k(table, *idxs, *wgts)


def kernel(query, reference_points, spatial_shapes, level_start_index,
           W_off, b_off, W_attn, b_attn, W_val, b_val, W_out, b_out):
    q2 = query.reshape(BS * NQ, C)
    rp2 = reference_points.reshape(BS * NQ, NL * 2)
    rpsel_np, bmask_np, cst_np = _np_selectors()
    wall = jnp.concatenate(
        [W_val, W_off[0::2], W_off[1::2], W_attn], axis=0).T
    half = jnp.float32(0.5)
    ball = jnp.concatenate(
        [b_val, b_off[0::2] - half, b_off[1::2] - half, b_attn]).reshape(1, -1)
    sel = jnp.asarray(rpsel_np)
    bmask = jnp.asarray(bmask_np)
    cst = jnp.asarray(cst_np)

    prep_out = pl.pallas_call(
        _prep_body,
        grid=(BS, NQT),
        in_specs=[
            pl.BlockSpec((QT, C), lambda b, i: (b * NQT + i, 0)),
            pl.BlockSpec((QT, NL * 2), lambda b, i: (b * NQT + i, 0)),
            pl.BlockSpec((C, C + 3 * NLANE), lambda b, i: (0, 0)),
            pl.BlockSpec((1, C + 3 * NLANE), lambda b, i: (0, 0)),
            pl.BlockSpec((NL * 2, 2 * NLANE), lambda b, i: (0, 0)),
            pl.BlockSpec((NLANE, NLANE), lambda b, i: (0, 0)),
            pl.BlockSpec((3, NLANE), lambda b, i: (0, 0)),
        ],
        out_specs=[pl.BlockSpec((QT, C), lambda b, i: (b * NQT + i, 0))]
        + [pl.BlockSpec((QT, NLANE), lambda b, i: (b * NQT + i, 0))] * 8,
        out_shape=[jax.ShapeDtypeStruct((BS * NQ, C), jnp.bfloat16)]
        + [jax.ShapeDtypeStruct((BS * NQ, NLANE), jnp.int32)] * 4
        + [jax.ShapeDtypeStruct((BS * NQ, NLANE), jnp.float32)] * 4,
    )(q2, rp2, wall, ball, sel, bmask, cst)
    value = prep_out[0]
    idxs = prep_out[1:5]
    wgts = prep_out[5:9]

    s_lo, s_hi = _sc_gather(value.reshape(NROWS, HD), idxs, wgts)

    perm = np.concatenate([np.concatenate([np.arange(h * HD, (h + 1) * HD, 2),
                                           np.arange(h * HD + 1, (h + 1) * HD, 2)])
                           for h in range(NH)])
    w_out_t = W_out.T[perm, :]

    out = pl.pallas_call(
        _post_body,
        grid=(BS * NQT,),
        in_specs=[
            pl.BlockSpec((QT, C // 2), lambda i: (i, 0)),
            pl.BlockSpec((QT, C // 2), lambda i: (i, 0)),
            pl.BlockSpec((QT, C), lambda i: (i, 0)),
            pl.BlockSpec((C // 2, C), lambda i: (0, 0)),
            pl.BlockSpec((C // 2, C), lambda i: (0, 0)),
            pl.BlockSpec((1, C), lambda i: (0, 0)),
        ],
        out_specs=pl.BlockSpec((QT, C), lambda i: (i, 0)),
        out_shape=jax.ShapeDtypeStruct((BS * NQ, C), jnp.float32),
    )(s_lo, s_hi, q2, w_out_t[:C // 2], w_out_t[C // 2:],
      b_out.reshape(1, C))

    return out.reshape(BS, NQ, C)
